# baseline (speedup 1.0000x reference)
import functools

import jax
import jax.numpy as jnp
from jax import lax
from jax.experimental import pallas as pl
from jax.experimental.pallas import tpu as pltpu
from jax.experimental.pallas import tpu_sc as plsc

_B, _T, _C = 1024, 200, 256
_NC, _NS = 2, 16
_NW = _NC * _NS
_R = _B * _T
_RPW = _R // _NW
_NR = 128
_CHUNKS = _RPW // _NR
_BUFW = _NR * _C

_mesh = plsc.VectorSubcoreMesh(core_axis_name="c", subcore_axis_name="s")


@functools.partial(
    pl.kernel,
    out_type=jax.ShapeDtypeStruct((_R * _C,), jnp.float32),
    mesh=_mesh,
    compiler_params=pltpu.CompilerParams(needs_layout_passes=False),
    scratch_types=[
        pltpu.VMEM((_RPW,), jnp.int32),
        pltpu.VMEM((_BUFW,), jnp.float32),
        pltpu.VMEM((_BUFW,), jnp.float32),
        pltpu.VMEM((_NR,), jnp.int32),
        pltpu.VMEM((_NR,), jnp.int32),
        pltpu.SemaphoreType.DMA,
        pltpu.SemaphoreType.DMA,
    ],
)
def _sc_onehot(x_hbm, out_hbm, xbuf, buf0, buf1, poss0, poss1, sem0, sem1):
    wid = lax.axis_index("s") * _NC + lax.axis_index("c")
    base = wid * _RPW
    pltpu.sync_copy(x_hbm.at[pl.ds(base, _RPW)], xbuf)

    iota = lax.iota(jnp.int32, 16)
    zeros_f = jnp.zeros((16,), jnp.float32)
    ones_f = jnp.ones((16,), jnp.float32)

    def _zero_body(i, carry):
        buf0[pl.ds(i * 16, 16)] = zeros_f
        buf1[pl.ds(i * 16, 16)] = zeros_f
        return carry

    lax.fori_loop(0, _BUFW // 16, _zero_body, jnp.int32(0))

    for g in range(_NR // 16):
        p_init = (g * 16 + iota) * _C
        poss0[pl.ds(g * 16, 16)] = p_init
        poss1[pl.ds(g * 16, 16)] = p_init

    def _do_chunk(c, buf, poss):
        for g in range(_NR // 16):
            rloc = c * _NR + g * 16 + iota
            t = lax.rem(rloc, _T)
            is_last = t == (_T - 1)
            gi = rloc + jnp.where(is_last, 0, 1)
            v = plsc.load_gather(xbuf, [gi]) + jnp.where(is_last, 1, 0)
            v = jnp.where(v > 255, 0, v)
            valid = v >= 0
            p = (g * 16 + iota) * _C + jnp.where(valid, v, 0)
            oldp = poss[pl.ds(g * 16, 16)]
            plsc.store_scatter(buf, [oldp], zeros_f)
            plsc.store_scatter(buf, [p], ones_f, mask=valid)
            poss[pl.ds(g * 16, 16)] = p

    def _start(c, buf, sem):
        off = base * _C + c * _BUFW
        pltpu.make_async_copy(buf, out_hbm.at[pl.ds(off, _BUFW)], sem).start()

    def _wait(c, buf, sem):
        off = base * _C + c * _BUFW
        pltpu.make_async_copy(buf, out_hbm.at[pl.ds(off, _BUFW)], sem).wait()

    _do_chunk(jnp.int32(0), buf0, poss0)
    _start(jnp.int32(0), buf0, sem0)
    _do_chunk(jnp.int32(1), buf1, poss1)
    _start(jnp.int32(1), buf1, sem1)

    def _loop(i, carry):
        ca = 2 * i + 2
        _wait(ca - 2, buf0, sem0)
        _do_chunk(ca, buf0, poss0)
        _start(ca, buf0, sem0)
        cb = 2 * i + 3
        _wait(cb - 2, buf1, sem1)
        _do_chunk(cb, buf1, poss1)
        _start(cb, buf1, sem1)
        return carry

    lax.fori_loop(0, (_CHUNKS - 2) // 2, _loop, jnp.int32(0))

    _wait(jnp.int32(_CHUNKS - 2), buf0, sem0)
    _wait(jnp.int32(_CHUNKS - 1), buf1, sem1)


def kernel(inputs):
    x = inputs.reshape(-1).astype(jnp.int32)
    out = _sc_onehot(x)
    return out.reshape(_B, _T, _C)


# speedup vs baseline: 1.0119x; 1.0119x over previous
import functools

import jax
import jax.numpy as jnp
from jax import lax
from jax.experimental import pallas as pl
from jax.experimental.pallas import tpu as pltpu
from jax.experimental.pallas import tpu_sc as plsc

_B, _T, _C = 1024, 200, 256
_NC, _NS = 2, 16
_NW = _NC * _NS
_R = _B * _T
_RPW = _R // _NW
_NR = 64
_NBUF = 4
_CHUNKS = _RPW // _NR
_BUFW = _NR * _C
_NG = _NR // 16

_mesh = plsc.VectorSubcoreMesh(core_axis_name="c", subcore_axis_name="s")


@functools.partial(
    pl.kernel,
    out_type=jax.ShapeDtypeStruct((_R * _C,), jnp.float32),
    mesh=_mesh,
    compiler_params=pltpu.CompilerParams(needs_layout_passes=False),
    scratch_types=(
        [pltpu.VMEM((_RPW,), jnp.int32)]
        + [pltpu.VMEM((_BUFW,), jnp.float32)] * _NBUF
        + [pltpu.VMEM((_NR,), jnp.int32)] * _NBUF
        + [pltpu.SemaphoreType.DMA] * _NBUF
    ),
)
def _sc_onehot(x_hbm, out_hbm, xbuf, *scr):
    bufs = scr[:_NBUF]
    posss = scr[_NBUF:2 * _NBUF]
    sems = scr[2 * _NBUF:]

    wid = lax.axis_index("s") * _NC + lax.axis_index("c")
    base = wid * _RPW
    pltpu.sync_copy(x_hbm.at[pl.ds(base, _RPW)], xbuf)

    iota = lax.iota(jnp.int32, 16)
    zeros_f = jnp.zeros((16,), jnp.float32)
    ones_f = jnp.ones((16,), jnp.float32)

    def _zero_body(i, carry):
        for s in range(_NBUF):
            bufs[s][pl.ds(i * 16, 16)] = zeros_f
        return carry

    lax.fori_loop(0, _BUFW // 16, _zero_body, jnp.int32(0))

    for g in range(_NG):
        p_init = (g * 16 + iota) * _C
        for s in range(_NBUF):
            posss[s][pl.ds(g * 16, 16)] = p_init

    def _do_chunk(c, buf, poss):
        for g in range(_NG):
            rloc = c * _NR + g * 16 + iota
            t = lax.rem(rloc, _T)
            is_last = t == (_T - 1)
            gi = rloc + jnp.where(is_last, 0, 1)
            v = plsc.load_gather(xbuf, [gi]) + jnp.where(is_last, 1, 0)
            v = jnp.where(v > 255, 0, v)
            valid = v >= 0
            p = (g * 16 + iota) * _C + jnp.where(valid, v, 0)
            oldp = poss[pl.ds(g * 16, 16)]
            plsc.store_scatter(buf, [oldp], zeros_f)
            plsc.store_scatter(buf, [p], ones_f, mask=valid)
            poss[pl.ds(g * 16, 16)] = p

    def _start(c, s):
        off = base * _C + c * _BUFW
        pltpu.make_async_copy(bufs[s], out_hbm.at[pl.ds(off, _BUFW)], sems[s]).start()

    def _wait(c, s):
        off = base * _C + c * _BUFW
        pltpu.make_async_copy(bufs[s], out_hbm.at[pl.ds(off, _BUFW)], sems[s]).wait()

    for s in range(_NBUF):
        _do_chunk(jnp.int32(s), bufs[s], posss[s])
        _start(jnp.int32(s), s)

    def _loop(i, carry):
        for s in range(_NBUF):
            c = _NBUF * i + _NBUF + s
            _wait(c - _NBUF, s)
            _do_chunk(c, bufs[s], posss[s])
            _start(c, s)
        return carry

    lax.fori_loop(0, (_CHUNKS - _NBUF) // _NBUF, _loop, jnp.int32(0))

    for s in range(_NBUF):
        _wait(jnp.int32(_CHUNKS - _NBUF + s), s)


def kernel(inputs):
    x = inputs.reshape(-1).astype(jnp.int32)
    out = _sc_onehot(x)
    return out.reshape(_B, _T, _C)


# speedup vs baseline: 3.5136x; 3.4724x over previous
import functools

import jax
import jax.numpy as jnp
from jax import lax
from jax.experimental import pallas as pl
from jax.experimental.pallas import tpu as pltpu
from jax.experimental.pallas import tpu_sc as plsc

_B, _T, _C = 1024, 200, 256
_NC, _NS = 2, 16
_NW = _NC * _NS
_R = _B * _T
_RPW = _R // _NW
_BPW = _B // _NW
_NR = 64
_NBUF = 4
_CHUNKS = _RPW // _NR
_NG = _NR // 16

_mesh = plsc.VectorSubcoreMesh(core_axis_name="c", subcore_axis_name="s")


@functools.partial(
    pl.kernel,
    out_type=jax.ShapeDtypeStruct((_R, _C), jnp.float32),
    mesh=_mesh,
    compiler_params=pltpu.CompilerParams(needs_layout_passes=False),
    scratch_types=(
        [pltpu.VMEM((_BPW, _T), jnp.int32)]
        + [pltpu.VMEM((_NR, _C), jnp.float32)] * _NBUF
        + [pltpu.VMEM((_NR,), jnp.int32)] * _NBUF
        + [pltpu.SemaphoreType.DMA] * _NBUF
    ),
)
def _sc_onehot(x_hbm, out_hbm, xbuf, *scr):
    bufs = scr[:_NBUF]
    posss = scr[_NBUF:2 * _NBUF]
    sems = scr[2 * _NBUF:]

    wid = lax.axis_index("s") * _NC + lax.axis_index("c")
    base = wid * _RPW
    pltpu.sync_copy(x_hbm.at[pl.ds(wid * _BPW, _BPW)], xbuf)

    iota = lax.iota(jnp.int32, 16)
    zeros_f = jnp.zeros((16,), jnp.float32)
    ones_f = jnp.ones((16,), jnp.float32)

    def _zero_body(i, carry):
        for s in range(_NBUF):
            for k in range(_C // 16):
                bufs[s][i, pl.ds(k * 16, 16)] = zeros_f
        return carry

    lax.fori_loop(0, _NR, _zero_body, jnp.int32(0))

    for s in range(_NBUF):
        for g in range(_NG):
            posss[s][pl.ds(g * 16, 16)] = jnp.zeros((16,), jnp.int32)

    def _do_chunk(c, buf, poss):
        for g in range(_NG):
            row = g * 16 + iota
            rloc = c * _NR + row
            t = lax.rem(rloc, _T)
            b = lax.div(rloc, _T)
            is_last = t == (_T - 1)
            tc = jnp.where(is_last, t, t + 1)
            v = plsc.load_gather(xbuf, [b, tc]) + jnp.where(is_last, 1, 0)
            v = jnp.where(v > 255, 0, v)
            valid = v >= 0
            vs = jnp.where(valid, v, 0)
            oldc = poss[pl.ds(g * 16, 16)]
            plsc.store_scatter(buf, [row, oldc], zeros_f)
            plsc.store_scatter(buf, [row, vs], ones_f, mask=valid)
            poss[pl.ds(g * 16, 16)] = vs

    def _start(c, s):
        r0 = base + c * _NR
        pltpu.make_async_copy(bufs[s], out_hbm.at[pl.ds(r0, _NR)], sems[s]).start()

    def _wait(c, s):
        r0 = base + c * _NR
        pltpu.make_async_copy(bufs[s], out_hbm.at[pl.ds(r0, _NR)], sems[s]).wait()

    for s in range(_NBUF):
        _do_chunk(jnp.int32(s), bufs[s], posss[s])
        _start(jnp.int32(s), s)

    def _loop(i, carry):
        for s in range(_NBUF):
            c = _NBUF * i + _NBUF + s
            _wait(c - _NBUF, s)
            _do_chunk(c, bufs[s], posss[s])
            _start(c, s)
        return carry

    lax.fori_loop(0, (_CHUNKS - _NBUF) // _NBUF, _loop, jnp.int32(0))

    for s in range(_NBUF):
        _wait(jnp.int32(_CHUNKS - _NBUF + s), s)


def kernel(inputs):
    x = inputs.astype(jnp.int32)
    out = _sc_onehot(x)
    return out.reshape(_B, _T, _C)
